# baseline (speedup 1.0000x reference)
import functools

import jax
import jax.numpy as jnp
from jax import lax
from jax.experimental import pallas as pl
from jax.experimental.pallas import tpu as pltpu
from jax.experimental.pallas import tpu_sc as plsc

_NC = 2
_NS = 16
_NW = _NC * _NS

_VOCAB = 200
_DIM = 32
_LANES = 16
_B_TOTAL = 16384 * 200
_PER_W = _B_TOTAL // _NW
_CHUNK = 640
_GROUPS = _CHUNK // _LANES
_NSLOTS = 4
_N_CHUNKS = _PER_W // _CHUNK
_N_ITERS = _N_CHUNKS // _NSLOTS

_mesh = plsc.VectorSubcoreMesh(
    core_axis_name="c", subcore_axis_name="s", num_cores=_NC, num_subcores=_NS
)


@functools.partial(
    pl.kernel,
    out_type=jax.ShapeDtypeStruct((_B_TOTAL, _DIM), jnp.float32),
    mesh=_mesh,
    scratch_types=[
        pltpu.VMEM((_VOCAB, _DIM), jnp.float32),
        pltpu.VMEM((_NSLOTS, _CHUNK), jnp.int32),
        pltpu.VMEM((_NSLOTS, _CHUNK, _DIM), jnp.float32),
        pltpu.SemaphoreType.DMA,
        pltpu.SemaphoreType.DMA,
    ],
    compiler_params=pltpu.CompilerParams(
        use_tc_tiling_on_sc=False, needs_layout_passes=False
    ),
)
def _emb_lookup(pos_hbm, table_hbm, out_hbm, tab_v, idx_v, rows_v, sem_i, sem_o):
    wid = lax.axis_index("s") * _NC + lax.axis_index("c")
    base = wid * _PER_W

    pltpu.sync_copy(table_hbm, tab_v)

    iota = lax.iota(jnp.int32, _LANES)
    cols = [(iota + c) & (_DIM - 1) for c in range(_DIM)]

    def start_idx(i, s):
        pltpu.async_copy(
            pos_hbm.at[pl.ds(base + i * _CHUNK, _CHUNK)], idx_v.at[s], sem_i
        )

    def wait_idx(s):
        pltpu.make_async_copy(
            pos_hbm.at[pl.ds(base, _CHUNK)], idx_v.at[s], sem_i
        ).wait()

    def start_out(i, s):
        pltpu.async_copy(
            rows_v.at[s],
            out_hbm.at[pl.ds(base + i * _CHUNK, _CHUNK), :],
            sem_o,
        )

    def wait_out(s):
        pltpu.make_async_copy(
            rows_v.at[s], out_hbm.at[pl.ds(base, _CHUNK), :], sem_o
        ).wait()

    for s in range(_NSLOTS):
        start_idx(s, s)

    @pl.loop(0, _N_ITERS)
    def _ring(j):
        for s in range(_NSLOTS):
            i = j * _NSLOTS + s
            wait_idx(s)

            @pl.when(j > 0)
            def _():
                wait_out(s)

            idx_slot = idx_v.at[s]
            rows_slot = rows_v.at[s]

            @pl.loop(0, _GROUPS)
            def _group(g):
                iv = idx_slot[pl.ds(g * _LANES, _LANES)]
                rowv = iota + g * _LANES
                for c in range(_DIM):
                    v = plsc.load_gather(tab_v, [iv, cols[c]])
                    plsc.store_scatter(rows_slot, [rowv, cols[c]], v)

            @pl.when(j < _N_ITERS - 1)
            def _():
                start_idx(i + _NSLOTS, s)

            start_out(i, s)

    for s in range(_NSLOTS):
        wait_out(s)


def kernel(positions, table):
    pos_flat = positions.reshape(_B_TOTAL)
    out = _emb_lookup(pos_flat, table)
    return out.reshape(positions.shape[0], positions.shape[1], _DIM)


# speedup vs baseline: 1.3490x; 1.3490x over previous
import functools

import jax
import jax.numpy as jnp
from jax import lax
from jax.experimental import pallas as pl
from jax.experimental.pallas import tpu as pltpu
from jax.experimental.pallas import tpu_sc as plsc

_NC = 2
_NS = 16
_NW = _NC * _NS

_VOCAB = 200
_DIM = 32
_LANES = 16
_B_TOTAL = 16384 * 200
_PER_W = _B_TOTAL // _NW
_CHUNK = 640
_GROUPS = _CHUNK // _LANES
_NSLOTS = 4
_N_CHUNKS = _PER_W // _CHUNK
_N_ITERS = _N_CHUNKS // _NSLOTS

_mesh = plsc.VectorSubcoreMesh(
    core_axis_name="c", subcore_axis_name="s", num_cores=_NC, num_subcores=_NS
)


@functools.partial(
    pl.kernel,
    out_type=jax.ShapeDtypeStruct((_B_TOTAL, _DIM), jnp.float32),
    mesh=_mesh,
    scratch_types=[
        pltpu.VMEM((_VOCAB, _DIM), jnp.float32),
        pltpu.VMEM((_NSLOTS, _CHUNK), jnp.int32),
        pltpu.VMEM((_NSLOTS, _CHUNK, _DIM), jnp.float32),
        pltpu.SemaphoreType.DMA,
        pltpu.SemaphoreType.DMA,
    ],
    compiler_params=pltpu.CompilerParams(
        use_tc_tiling_on_sc=False, needs_layout_passes=False
    ),
)
def _emb_lookup(pos_hbm, table_hbm, out_hbm, tab_v, idx_v, rows_v, sem_i, sem_o):
    wid = lax.axis_index("s") * _NC + lax.axis_index("c")
    base = wid * _PER_W

    pltpu.sync_copy(table_hbm, tab_v)

    iota = lax.iota(jnp.int32, _LANES)
    cols = [(iota + c) & (_DIM - 1) for c in range(_DIM)]

    def start_idx(i, s):
        pltpu.async_copy(
            pos_hbm.at[pl.ds(base + i * _CHUNK, _CHUNK)], idx_v.at[s], sem_i
        )

    def wait_idx(s):
        pltpu.make_async_copy(
            pos_hbm.at[pl.ds(base, _CHUNK)], idx_v.at[s], sem_i
        ).wait()

    def start_out(i, s):
        pltpu.async_copy(
            rows_v.at[s],
            out_hbm.at[pl.ds(base + i * _CHUNK, _CHUNK), :],
            sem_o,
        )

    def wait_out(s):
        pltpu.make_async_copy(
            rows_v.at[s], out_hbm.at[pl.ds(base, _CHUNK), :], sem_o
        ).wait()

    for s in range(_NSLOTS):
        start_idx(s, s)

    @pl.loop(0, _N_ITERS)
    def _ring(j):
        for s in range(_NSLOTS):
            i = j * _NSLOTS + s
            wait_idx(s)

            @pl.when(j > 0)
            def _():
                wait_out(s)

            idx_slot = idx_v.at[s]
            rows_slot = rows_v.at[s]

            @pl.loop(0, _GROUPS)
            def _group(g):
                iv = idx_slot[pl.ds(g * _LANES, _LANES)]
                rowv = iota + g * _LANES
                lag = 4
                vs = [None] * _DIM
                for c in range(_DIM):
                    vs[c] = plsc.load_gather(tab_v, [iv, cols[c]])
                    if c >= lag:
                        plsc.store_scatter(
                            rows_slot, [rowv, cols[c - lag]], vs[c - lag]
                        )
                for c in range(_DIM - lag, _DIM):
                    plsc.store_scatter(rows_slot, [rowv, cols[c]], vs[c])

            @pl.when(j < _N_ITERS - 1)
            def _():
                start_idx(i + _NSLOTS, s)

            start_out(i, s)

    for s in range(_NSLOTS):
        wait_out(s)


def kernel(positions, table):
    pos_flat = positions.reshape(_B_TOTAL)
    out = _emb_lookup(pos_flat, table)
    return out.reshape(positions.shape[0], positions.shape[1], _DIM)
